# Initial kernel scaffold; baseline (speedup 1.0000x reference)
import jax
import jax.numpy as jnp
from jax.experimental import pallas as pl


def kernel(ego_embeddings, A_rows, A_cols, A_vals, proj1_rows, proj1_cols, proj1_vals, proj2_vals, lib1_rows, lib1_cols, lib1_vals, lib2_vals, W1, b1, W2, b2):
    raise NotImplementedError("write your pallas kernel here")


# speedup vs baseline: 1.6746x; 1.6746x over previous
import jax
import jax.numpy as jnp
from jax import lax
from jax.experimental import pallas as pl
from jax.experimental.pallas import tpu as pltpu
from jax.experimental.pallas import tpu_sc as plsc

N = 10000
D = 128
E = 320000
H = 5000
P = 160000

NC = 2
NS = 16
HD = D // NC
CHUNK = 128
ZROWS = 40
WROWS = 80


def _cdiv(a, b):
    return (a + b - 1) // b


def _sc_body(ego2, a_rows, a_cols, a_vals,
             p1r, p1c, p1v, p2v,
             l1r, l1c, l1v, l2v,
             out2,
             idx_d, idx_s, vals_v, rows_v, zbuf, widx,
             acc_sh, h_sh):
    c = lax.axis_index("c")
    s = lax.axis_index("s")

    zero16 = jnp.zeros((16,), jnp.float32)

    @pl.loop(0, ZROWS)
    def _(r):
        for j in range(HD // 16):
            zbuf[r, pl.ds(j * 16, 16)] = zero16

    n_chunks_acc = N // ZROWS
    n_chunks_h = H // ZROWS

    @pl.loop(0, _cdiv(n_chunks_acc, NS))
    def _(it):
        ch = it * NS + s

        @pl.when(ch < n_chunks_acc)
        def _():
            pltpu.sync_copy(zbuf, acc_sh.at[pl.ds(ch * ZROWS, ZROWS)])

    def zero_h():
        @pl.loop(0, _cdiv(n_chunks_h, NS))
        def _(it):
            ch = it * NS + s

            @pl.when(ch < n_chunks_h)
            def _():
                pltpu.sync_copy(zbuf, h_sh.at[pl.ds(ch * ZROWS, ZROWS)])

    zero_h()
    plsc.subcore_barrier()

    def scale_rows():
        @pl.loop(0, CHUNK // 16)
        def _(g):
            vv = vals_v[pl.ds(g * 16, 16)]
            for k in range(16):
                sv = vv[k]
                i = g * 16 + k
                for j in range(HD // 16):
                    sl = pl.ds(j * 16, 16)
                    rows_v[i, sl] = rows_v[i, sl] * sv

    def process(src_ref, dst_sh, rows_hbm, cols_hbm, vals_hbm, n_edges,
                stacked):
        n_ch = n_edges // CHUNK

        @pl.loop(0, _cdiv(n_ch, NS))
        def _(it):
            ch = it * NS + s

            @pl.when(ch < n_ch)
            def _():
                base = ch * CHUNK
                pltpu.sync_copy(rows_hbm.at[pl.ds(base, CHUNK)], idx_d)
                pltpu.sync_copy(cols_hbm.at[pl.ds(base, CHUNK)], idx_s)
                pltpu.sync_copy(vals_hbm.at[pl.ds(base, CHUNK)], vals_v)
                if stacked:
                    for g in range(CHUNK // 16):
                        sl = pl.ds(g * 16, 16)
                        idx_s[sl] = idx_s[sl] * 2 + c
                pltpu.sync_copy(src_ref.at[idx_s], rows_v)
                scale_rows()
                pltpu.sync_copy(rows_v, dst_sh.at[idx_d], add=True)

    process(ego2, acc_sh, a_rows, a_cols, a_vals, E, True)

    process(ego2, h_sh, p1r, p1c, p1v, P, True)
    plsc.subcore_barrier()

    process(h_sh, acc_sh, p1c, p1r, p2v, P, False)
    plsc.subcore_barrier()

    zero_h()
    plsc.subcore_barrier()
    process(ego2, h_sh, l1r, l1c, l1v, P, True)
    plsc.subcore_barrier()
    process(h_sh, acc_sh, l1c, l1r, l2v, P, False)
    plsc.subcore_barrier()

    n_wch = N // WROWS
    iota16 = lax.iota(jnp.int32, 16)

    @pl.loop(0, _cdiv(n_wch, NS))
    def _(it):
        ch = it * NS + s

        @pl.when(ch < n_wch)
        def _():
            base = ch * WROWS
            pltpu.sync_copy(acc_sh.at[pl.ds(base, WROWS)],
                            rows_v.at[pl.ds(0, WROWS)])
            for g in range(WROWS // 16):
                widx[pl.ds(g * 16, 16)] = (iota16 + (base + g * 16)) * 2 + c
            pltpu.sync_copy(rows_v.at[pl.ds(0, WROWS)], out2.at[widx])


def _sc_aggregate(ego2, a_rows, a_cols, a_vals, p1r, p1c, p1v, p2v,
                  l1r, l1c, l1v, l2v):
    mesh = plsc.VectorSubcoreMesh(core_axis_name="c", subcore_axis_name="s")
    f = pl.kernel(
        _sc_body,
        out_type=jax.ShapeDtypeStruct((2 * N, HD), jnp.float32),
        mesh=mesh,
        compiler_params=pltpu.CompilerParams(use_tc_tiling_on_sc=False),
        scratch_types=(
            pltpu.VMEM((CHUNK,), jnp.int32),
            pltpu.VMEM((CHUNK,), jnp.int32),
            pltpu.VMEM((CHUNK,), jnp.float32),
            pltpu.VMEM((CHUNK, HD), jnp.float32),
            pltpu.VMEM((ZROWS, HD), jnp.float32),
            pltpu.VMEM((WROWS,), jnp.int32),
            pltpu.VMEM_SHARED((N, HD), jnp.float32),
            pltpu.VMEM_SHARED((H, HD), jnp.float32),
        ),
    )
    return f(ego2, a_rows, a_cols, a_vals, p1r, p1c, p1v, p2v,
             l1r, l1c, l1v, l2v)


BM = 1000


def _mlp_body(ego_ref, side_ref, w1_ref, b1_ref, w2_ref, b2_ref, o_ref):
    ego = ego_ref[...]
    side = side_ref[...]
    dn = (((1,), (1,)), ((), ()))
    x1 = lax.dot_general(ego + side, w1_ref[...], dn,
                         preferred_element_type=jnp.float32) + b1_ref[...]
    x2 = lax.dot_general(ego * side, w2_ref[...], dn,
                         preferred_element_type=jnp.float32) + b2_ref[...]
    o_ref[...] = (jnp.where(x1 > 0, x1, 0.01 * x1)
                  + jnp.where(x2 > 0, x2, 0.01 * x2))


def _mlp(ego, side, W1, b1, W2, b2):
    grid = (N // BM,)
    row_spec = pl.BlockSpec((BM, D), lambda i: (i, 0))
    full_spec = pl.BlockSpec((D, D), lambda i: (0, 0))
    bias_spec = pl.BlockSpec((1, D), lambda i: (0, 0))
    return pl.pallas_call(
        _mlp_body,
        grid=grid,
        in_specs=[row_spec, row_spec,
                  full_spec, bias_spec, full_spec, bias_spec],
        out_specs=row_spec,
        out_shape=jax.ShapeDtypeStruct((N, D), jnp.float32),
    )(ego, side, W1, b1.reshape(1, D), W2, b2.reshape(1, D))


def kernel(ego_embeddings, A_rows, A_cols, A_vals,
           proj1_rows, proj1_cols, proj1_vals, proj2_vals,
           lib1_rows, lib1_cols, lib1_vals, lib2_vals,
           W1, b1, W2, b2):
    i32 = jnp.int32
    ego2 = ego_embeddings.reshape(N, NC, HD).reshape(NC * N, HD)
    side2 = _sc_aggregate(
        ego2,
        A_rows.astype(i32), A_cols.astype(i32), A_vals,
        proj1_rows.astype(i32), proj1_cols.astype(i32), proj1_vals, proj2_vals,
        lib1_rows.astype(i32), lib1_cols.astype(i32), lib1_vals, lib2_vals)
    side = side2.reshape(N, D)
    return _mlp(ego_embeddings, side, W1, b1, W2, b2)
